# Initial kernel scaffold; baseline (speedup 1.0000x reference)
import jax
import jax.numpy as jnp
from jax.experimental import pallas as pl


def kernel(node_feats, g_feats, segment_ids, W1, b1, W2, b2, Wih, Whh, bih, bhh):
    raise NotImplementedError("write your pallas kernel here")


# speedup vs baseline: 6.1031x; 6.1031x over previous
import jax
import jax.numpy as jnp
from jax import lax
from jax.experimental import pallas as pl
from jax.experimental.pallas import tpu as pltpu
from jax.experimental.pallas import tpu_sc as plsc

N = 50000
B = 1024
F = 256
L = 16
NC = 2
NS = 16
NW = NC * NS
SEGW = B // NW
COLS = F + L
CHUNK = 112


def _sc_body(x_hbm, seg_hbm, g_hbm, w1a_hbm, w1b_hbm, b1_hbm, s_out,
             xbuf, segall, acc, csbuf, cbuf, w1abuf, w1bbuf, b1buf,
             c_shared):
    cid = lax.axis_index("c")
    sid = lax.axis_index("s")
    w = cid * NS + sid
    iot = lax.iota(jnp.int32, L)
    fzeros = jnp.zeros((L,), jnp.float32)
    lane0 = (iot == 0).astype(jnp.float32)

    pltpu.sync_copy(w1a_hbm, w1abuf)
    pltpu.sync_copy(w1b_hbm, w1bbuf)
    pltpu.sync_copy(b1_hbm, b1buf)
    pltpu.sync_copy(seg_hbm, segall)

    def zbody(r, _):
        for j in range(COLS // L):
            acc[r, pl.ds(j * L, L)] = fzeros
        return _
    lax.fori_loop(0, SEGW, zbody, None)

    pltpu.sync_copy(g_hbm.at[pl.ds(sid * 64, 64)], xbuf.at[pl.ds(0, 64)])
    wa = [w1abuf[pl.ds(j * L, L)] for j in range(L)]
    b1s = b1buf[pl.ds(0, L)][0]
    for g in range(4):
        cacc = fzeros
        for jrow in range(L):
            r = g * L + jrow
            xv = [jnp.maximum(xbuf[r, pl.ds(j * L, L)], 0.0) for j in range(L)]
            p = [xv[j] * wa[j] for j in range(L)]
            while len(p) > 1:
                p = [p[2 * i] + p[2 * i + 1] for i in range(len(p) // 2)]
            u = jnp.sum(p[0]) + b1s
            cacc = jnp.where(iot == jrow, u + fzeros, cacc)
        csbuf[pl.ds(g * L, L)] = cacc
    pltpu.sync_copy(csbuf, c_shared.at[pl.ds(sid * 64, 64)])

    plsc.subcore_barrier()
    pltpu.sync_copy(c_shared, cbuf)

    wb = [w1bbuf[pl.ds(j * L, L)] for j in range(L)]

    def first_ge(t):
        lo = jnp.int32(0)
        for step in (4096, 256, 16):
            probes = lo + iot * step
            pv = plsc.load_gather(segall, [jnp.minimum(probes, N - 1)])
            pv = jnp.where(probes < N, pv, jnp.int32(2 ** 30))
            c = jnp.sum((pv < t).astype(jnp.int32))
            lo = lo + jnp.maximum(c - 1, 0) * step
        probes = jnp.minimum(lo + iot, N - 1)
        pv = plsc.load_gather(segall, [probes])
        pv = jnp.where(lo + iot < N, pv, jnp.int32(2 ** 30))
        return lo + jnp.sum((pv < t).astype(jnp.int32))

    wlo = w * SEGW
    rs = first_ge(wlo)
    re = first_ge(wlo + SEGW)

    lo16 = (rs // L) * L
    hi16 = ((re + L - 1) // L) * L
    nch = (hi16 - lo16 + CHUNK - 1) // CHUNK

    def chunk_body(i, _):
        logical = lo16 + i * CHUNK
        row0 = pl.multiple_of(jnp.minimum(logical, N - CHUNK), L)
        pltpu.sync_copy(x_hbm.at[pl.ds(row0, CHUNK)], xbuf)

        def group(gi, _):
            base = row0 + gi * L
            svec = segall[pl.ds(base, L)]
            rowid = base + iot
            m = (svec >= wlo) & (svec < wlo + SEGW) & (rowid >= logical)
            mf = m.astype(jnp.float32)
            cg = plsc.load_gather(cbuf, [jnp.clip(svec, 0, B - 1)])
            lsv = jnp.clip(svec - wlo, 0, SEGW - 1)
            for jrow in range(L):
                r = gi * L + jrow
                xv = [xbuf[r, pl.ds(j * L, L)] for j in range(L)]
                p = [xv[j] * wb[j] for j in range(L)]
                while len(p) > 1:
                    p = [p[2 * i2] + p[2 * i2 + 1] for i2 in range(len(p) // 2)]
                zc = jnp.sum(p[0]) + cg[jrow]
                z = jnp.where(zc > 0, zc, zc * jnp.float32(0.01))
                ev = jnp.exp(z + fzeros) * (mf[jrow] + fzeros)
                ls = lsv[jrow]
                for j in range(L):
                    acc[ls, pl.ds(j * L, L)] = acc[ls, pl.ds(j * L, L)] + xv[j] * ev
                acc[ls, pl.ds(F, L)] = acc[ls, pl.ds(F, L)] + ev * lane0
            return _

        lax.fori_loop(0, CHUNK // L, group, None)
        return _

    lax.fori_loop(0, nch, chunk_body, None)

    pltpu.sync_copy(acc, s_out.at[pl.ds(w * SEGW, SEGW)])


def _finish_body(s_ref, g_ref, w2_ref, b2_ref, wih_ref, whh_ref,
                 bih_ref, bhh_ref, out_ref):
    sa = s_ref[...]
    lanes = lax.broadcasted_iota(jnp.int32, (1, COLS), 1)
    dmask = (lanes == F).astype(jnp.float32)
    d = jnp.sum(sa * dmask, axis=1, keepdims=True)
    s = sa[:, 0:F]
    occ = (d > 0).astype(jnp.float32)
    inv = jnp.where(d > 0, 1.0 / jnp.where(d > 0, d, 1.0), 0.0)
    p = s * inv
    dims = (((1,), (1,)), ((), ()))
    g_repr = lax.dot_general(p, w2_ref[...], dims,
                             preferred_element_type=jnp.float32)
    g_repr = g_repr + occ * b2_ref[...][None, :]
    context = jnp.where(g_repr > 0, g_repr,
                        jnp.exp(jnp.minimum(g_repr, 0.0)) - 1.0)
    gi = lax.dot_general(context, wih_ref[...], dims,
                         preferred_element_type=jnp.float32) + bih_ref[...][None, :]
    gh = lax.dot_general(g_ref[...], whh_ref[...], dims,
                         preferred_element_type=jnp.float32) + bhh_ref[...][None, :]
    i_r, i_z, i_n = gi[:, 0:F], gi[:, F:2 * F], gi[:, 2 * F:3 * F]
    h_r, h_z, h_n = gh[:, 0:F], gh[:, F:2 * F], gh[:, 2 * F:3 * F]

    def sig(t):
        return 1.0 / (1.0 + jnp.exp(-t))

    r = sig(i_r + h_r)
    u = sig(i_z + h_z)
    t2 = i_n + r * h_n
    n = 2.0 * sig(2.0 * t2) - 1.0
    out_ref[...] = (1.0 - u) * n + u * g_ref[...]


@jax.jit
def kernel(node_feats, g_feats, segment_ids, W1, b1, W2, b2, Wih, Whh, bih, bhh):
    w1a = W1[0, :F]
    w1b = W1[0, F:]
    b1v = jnp.broadcast_to(b1, (L,)).astype(jnp.float32)
    seg = segment_ids.astype(jnp.int32)

    mesh = plsc.VectorSubcoreMesh(core_axis_name="c", subcore_axis_name="s")
    sc = pl.kernel(
        _sc_body,
        out_type=jax.ShapeDtypeStruct((B, COLS), jnp.float32),
        mesh=mesh,
        compiler_params=pltpu.CompilerParams(needs_layout_passes=False),
        scratch_types=[
            pltpu.VMEM((CHUNK, F), jnp.float32),
            pltpu.VMEM((N,), jnp.int32),
            pltpu.VMEM((SEGW, COLS), jnp.float32),
            pltpu.VMEM((64,), jnp.float32),
            pltpu.VMEM((B,), jnp.float32),
            pltpu.VMEM((F,), jnp.float32),
            pltpu.VMEM((F,), jnp.float32),
            pltpu.VMEM((L,), jnp.float32),
            pltpu.VMEM_SHARED((B,), jnp.float32),
        ],
    )
    s2 = sc(node_feats, seg, g_feats, w1a, w1b, b1v)

    h_new = pl.pallas_call(
        _finish_body,
        out_shape=jax.ShapeDtypeStruct((B, F), jnp.float32),
    )(s2, g_feats, W2, b2, Wih, Whh, bih, bhh)
    return h_new
